# Initial kernel scaffold; baseline (speedup 1.0000x reference)
import functools
import math

import jax
import jax.numpy as jnp
from jax import lax
from jax.experimental import pallas as pl
from jax.experimental.pallas import tpu as pltpu
from jax.experimental.pallas import tpu_sc as plsc

_MIN_KEPT = 100000
_THRESH = 0.7
_TAU0 = float(-math.log(_THRESH))

_BH = 64
_NT = 16
_LN = 16


def _ce_body(pred_ref, tgt_ref, out_ref):
    x = pred_ref[0]
    t = tgt_ref[0]
    m = jnp.max(x, axis=0)
    s = jnp.sum(jnp.exp(x - m[None]), axis=0)
    cls = lax.broadcasted_iota(jnp.int32, x.shape, 0)
    xt = jnp.sum(jnp.where(cls == t[None], x, 0.0), axis=0)
    out_ref[0] = (m - xt) + jnp.log(s)


def _ce_losses(predict, target):
    B, C, H, W = predict.shape
    return pl.pallas_call(
        _ce_body,
        grid=(B, H // _BH),
        in_specs=[
            pl.BlockSpec((1, C, _BH, W), lambda b, h: (b, 0, h, 0)),
            pl.BlockSpec((1, _BH, W), lambda b, h: (b, h, 0)),
        ],
        out_specs=pl.BlockSpec((1, _BH, W), lambda b, h: (b, h, 0)),
        out_shape=jax.ShapeDtypeStruct((B, H, W), jnp.float32),
    )(predict, target)


@functools.lru_cache(maxsize=None)
def _make_select(n):
    chunk = n // _NT
    iters = chunk // _LN
    kept = min(_MIN_KEPT, n - 1)
    rank = float(n - 1 - kept)
    kept_f = float(kept)
    mesh = plsc.VectorSubcoreMesh(
        core_axis_name="c", subcore_axis_name="s", num_cores=1)

    @functools.partial(
        pl.kernel,
        out_type=jax.ShapeDtypeStruct((_LN,), jnp.float32),
        mesh=mesh,
        scratch_types=[
            pltpu.VMEM((chunk,), jnp.float32),
            pltpu.VMEM_SHARED((_NT * 48,), jnp.float32),
            pltpu.VMEM((_NT * 48,), jnp.float32),
            pltpu.VMEM((48,), jnp.float32),
            pltpu.VMEM((_LN,), jnp.float32),
        ],
    )
    def sel(l_hbm, out_hbm, buf, stage, stage_l, pub, obuf):
        wid = lax.axis_index("s")
        zeros = jnp.zeros((_LN,), jnp.float32)
        lane = lax.broadcasted_iota(jnp.int32, (_LN,), 0)

        pltpu.sync_copy(l_hbm.at[pl.ds(wid * chunk, chunk)], buf)

        def vchunk(j):
            return buf[pl.ds(pl.multiple_of(j * _LN, _LN), _LN)]

        def p1(j, carry):
            g, e, s = carry
            v = vchunk(j)
            g = g + jnp.where(v > _TAU0, 1.0, 0.0)
            e = e + jnp.where(v >= _TAU0, 1.0, 0.0)
            s = s + jnp.where(v > _TAU0, v, 0.0)
            return g, e, s

        g, e, s = lax.fori_loop(0, iters, p1, (zeros, zeros, zeros))

        def combine3(a, b, c):
            pub[pl.ds(0, _LN)] = a
            pub[pl.ds(16, _LN)] = b
            pub[pl.ds(32, _LN)] = c
            pltpu.sync_copy(pub, stage.at[pl.ds(wid * 48, 48)])
            plsc.subcore_barrier()
            pltpu.sync_copy(stage, stage_l)
            ta, tb, tc = zeros, zeros, zeros
            for t in range(_NT):
                ta = ta + stage_l[pl.ds(t * 48, _LN)]
                tb = tb + stage_l[pl.ds(t * 48 + 16, _LN)]
                tc = tc + stage_l[pl.ds(t * 48 + 32, _LN)]
            plsc.subcore_barrier()
            return jnp.sum(ta), jnp.sum(tb), jnp.sum(tc)

        c_gt, c_ge, s_gt = combine3(g, e, s)
        ans0 = jnp.where(c_gt > 0.0, s_gt / jnp.maximum(c_gt, 1.0), 0.0)

        def fallback(_):
            def bit_round(i, p):
                t_pat = p | lax.shift_left(jnp.int32(1), jnp.int32(30) - i)

                def cbody(j, acc):
                    vb = plsc.bitcast(vchunk(j), jnp.int32)
                    return acc + jnp.where(vb < t_pat, 1.0, 0.0)

                cl = lax.fori_loop(0, iters, cbody, zeros)
                total, _, _ = combine3(cl, zeros, zeros)
                return jnp.where(total <= rank, t_pat, p)

            p = lax.fori_loop(0, 31, bit_round, jnp.int32(0))

            def fbody(j, carry):
                g2, s2 = carry
                v = vchunk(j)
                keep = plsc.bitcast(v, jnp.int32) > p
                return (g2 + jnp.where(keep, 1.0, 0.0),
                        s2 + jnp.where(keep, v, 0.0))

            g2, s2 = lax.fori_loop(0, iters, fbody, (zeros, zeros))
            c_d, s_d, _ = combine3(g2, s2, zeros)
            return jnp.where(c_d > 0.0, s_d / jnp.maximum(c_d, 1.0), 0.0)

        ans = lax.cond(c_ge <= kept_f, fallback, lambda _: ans0, None)

        @pl.when(wid == 0)
        def _():
            obuf[...] = jnp.where(lane >= 0, ans, 0.0)
            pltpu.sync_copy(obuf, out_hbm)

    return sel


def kernel(predict, target):
    ce = _ce_losses(predict, target.astype(jnp.int32))
    flat = ce.reshape(-1)
    out = _make_select(flat.shape[0])(flat)
    return out[0]


# speedup vs baseline: 22.0879x; 22.0879x over previous
import functools
import math

import jax
import jax.numpy as jnp
from jax import lax
from jax.experimental import pallas as pl
from jax.experimental.pallas import tpu as pltpu
from jax.experimental.pallas import tpu_sc as plsc

_MIN_KEPT = 100000
_THRESH = 0.7
_TAU0 = float(-math.log(_THRESH))

_BH = 64
_NT = 16
_LN = 16


def _ce_body(pred_ref, tgt_ref, out_ref):
    x = pred_ref[0]
    t = tgt_ref[0]
    m = jnp.max(x, axis=0)
    s = jnp.sum(jnp.exp(x - m[None]), axis=0)
    cls = lax.broadcasted_iota(jnp.int32, x.shape, 0)
    xt = jnp.sum(jnp.where(cls == t[None], x, 0.0), axis=0)
    out_ref[0] = (m - xt) + jnp.log(s)


def _ce_losses(predict, target):
    B, C, H, W = predict.shape
    return pl.pallas_call(
        _ce_body,
        grid=(B, H // _BH),
        in_specs=[
            pl.BlockSpec((1, C, _BH, W), lambda b, h: (b, 0, h, 0)),
            pl.BlockSpec((1, _BH, W), lambda b, h: (b, h, 0)),
        ],
        out_specs=pl.BlockSpec((1, _BH, W), lambda b, h: (b, h, 0)),
        out_shape=jax.ShapeDtypeStruct((B, H, W), jnp.float32),
    )(predict, target)


@functools.lru_cache(maxsize=None)
def _make_select(n):
    chunk = n // _NT
    iters = chunk // _LN
    kept = min(_MIN_KEPT, n - 1)
    rank = float(n - 1 - kept)
    kept_f = float(kept)
    mesh = plsc.VectorSubcoreMesh(
        core_axis_name="c", subcore_axis_name="s", num_cores=1)

    @functools.partial(
        pl.kernel,
        out_type=jax.ShapeDtypeStruct((_LN,), jnp.float32),
        mesh=mesh,
        compiler_params=pltpu.CompilerParams(needs_layout_passes=False),
        scratch_types=[
            pltpu.VMEM((chunk,), jnp.float32),
            pltpu.VMEM_SHARED((_NT * 48,), jnp.float32),
            pltpu.VMEM((_NT * 48,), jnp.float32),
            pltpu.VMEM((48,), jnp.float32),
            pltpu.VMEM((_LN,), jnp.float32),
            pltpu.VMEM((_LN,), jnp.float32),
        ],
    )
    def sel(l_hbm, out_hbm, buf, stage, stage_l, pub, red, obuf):
        wid = lax.axis_index("s")
        zeros = jnp.zeros((_LN,), jnp.float32)
        lane = lax.broadcasted_iota(jnp.int32, (_LN,), 0)

        pltpu.sync_copy(l_hbm.at[pl.ds(wid * chunk, chunk)], buf)

        def vchunk(j):
            return buf[pl.ds(pl.multiple_of(j * _LN, _LN), _LN)]

        def p1(j, carry):
            g, e, s = carry
            v = vchunk(j)
            g = g + jnp.where(v > _TAU0, 1.0, 0.0)
            e = e + jnp.where(v >= _TAU0, 1.0, 0.0)
            s = s + jnp.where(v > _TAU0, v, 0.0)
            return g, e, s

        g, e, s = lax.fori_loop(0, iters, p1, (zeros, zeros, zeros))

        def vec_to_scalar(v):
            acc = v[0]
            for i in range(1, _LN):
                acc = acc + v[i]
            return acc

        def combine3(a, b, c):
            pub[pl.ds(0, _LN)] = a
            pub[pl.ds(16, _LN)] = b
            pub[pl.ds(32, _LN)] = c
            pltpu.sync_copy(pub, stage.at[pl.ds(wid * 48, 48)])
            plsc.subcore_barrier()
            pltpu.sync_copy(stage, stage_l)
            ta, tb, tc = zeros, zeros, zeros
            for t in range(_NT):
                ta = ta + stage_l[pl.ds(t * 48, _LN)]
                tb = tb + stage_l[pl.ds(t * 48 + 16, _LN)]
                tc = tc + stage_l[pl.ds(t * 48 + 32, _LN)]
            plsc.subcore_barrier()
            return vec_to_scalar(ta), vec_to_scalar(tb), vec_to_scalar(tc)

        c_gt, c_ge, s_gt = combine3(g, e, s)

        def fallback(_):
            def bit_round(i, p):
                t_pat = p | lax.shift_left(jnp.int32(1), jnp.int32(30) - i)

                def cbody(j, acc):
                    vb = plsc.bitcast(vchunk(j), jnp.int32)
                    return acc + jnp.where(vb < t_pat, 1.0, 0.0)

                cl = lax.fori_loop(0, iters, cbody, zeros)
                total, _, _ = combine3(cl, zeros, zeros)
                return jnp.where(total <= rank, t_pat, p)

            p = lax.fori_loop(0, 31, bit_round, jnp.int32(0))

            def fbody(j, carry):
                g2, s2 = carry
                v = vchunk(j)
                keep = plsc.bitcast(v, jnp.int32) > p
                return (g2 + jnp.where(keep, 1.0, 0.0),
                        s2 + jnp.where(keep, v, 0.0))

            g2, s2 = lax.fori_loop(0, iters, fbody, (zeros, zeros))
            c_d, s_d, _ = combine3(g2, s2, zeros)
            return s_d, c_d

        s_sel, c_sel = lax.cond(
            c_ge <= kept_f, fallback, lambda _: (s_gt, c_gt), None)

        @pl.when(wid == 0)
        def _():
            obuf[...] = jnp.where(
                lane == 0, s_sel, jnp.where(lane == 1, c_sel, 0.0))
            pltpu.sync_copy(obuf, out_hbm)

    return sel


def kernel(predict, target):
    ce = _ce_losses(predict, target.astype(jnp.int32))
    flat = ce.reshape(-1)
    out = _make_select(flat.shape[0])(flat)
    s, c = out[0], out[1]
    return jnp.where(c > 0.0, s / jnp.maximum(c, 1.0), 0.0)


# speedup vs baseline: 23.9670x; 1.0851x over previous
import functools
import math

import jax
import jax.numpy as jnp
from jax import lax
from jax.experimental import pallas as pl
from jax.experimental.pallas import tpu as pltpu
from jax.experimental.pallas import tpu_sc as plsc

_MIN_KEPT = 100000
_THRESH = 0.7
_TAU0 = float(-math.log(_THRESH))

_BH = 64
_NC = 2
_NT = 16
_LN = 16


def _ce_body(pred_ref, tgt_ref, out_ref):
    x = pred_ref[0]
    t = tgt_ref[0]
    m = jnp.max(x, axis=0)
    s = jnp.sum(jnp.exp(x - m[None]), axis=0)
    cls = lax.broadcasted_iota(jnp.int32, x.shape, 0)
    xt = jnp.sum(jnp.where(cls == t[None], x, 0.0), axis=0)
    out_ref[0] = (m - xt) + jnp.log(s)


def _ce_losses(predict, target):
    B, C, H, W = predict.shape
    return pl.pallas_call(
        _ce_body,
        grid=(B, H // _BH),
        in_specs=[
            pl.BlockSpec((1, C, _BH, W), lambda b, h: (b, 0, h, 0)),
            pl.BlockSpec((1, _BH, W), lambda b, h: (b, h, 0)),
        ],
        out_specs=pl.BlockSpec((1, _BH, W), lambda b, h: (b, h, 0)),
        out_shape=jax.ShapeDtypeStruct((B, H, W), jnp.float32),
    )(predict, target)


@functools.lru_cache(maxsize=None)
def _make_phase1(n):
    nw = _NC * _NT
    chunk = n // nw
    iters = chunk // _LN
    mesh = plsc.VectorSubcoreMesh(
        core_axis_name="c", subcore_axis_name="s", num_cores=_NC)

    @functools.partial(
        pl.kernel,
        out_type=jax.ShapeDtypeStruct((nw, 48), jnp.float32),
        mesh=mesh,
        compiler_params=pltpu.CompilerParams(needs_layout_passes=False),
        scratch_types=[
            pltpu.VMEM((chunk,), jnp.float32),
            pltpu.VMEM((48,), jnp.float32),
        ],
    )
    def phase1(l_hbm, out_hbm, buf, pub):
        wid = lax.axis_index("s") * _NC + lax.axis_index("c")
        zeros = jnp.zeros((_LN,), jnp.float32)

        pltpu.sync_copy(l_hbm.at[pl.ds(wid * chunk, chunk)], buf)

        def body(j, carry):
            g, e, s = carry
            v = buf[pl.ds(pl.multiple_of(j * _LN, _LN), _LN)]
            g = g + jnp.where(v > _TAU0, 1.0, 0.0)
            e = e + jnp.where(v >= _TAU0, 1.0, 0.0)
            s = s + jnp.where(v > _TAU0, v, 0.0)
            return g, e, s

        g, e, s = lax.fori_loop(0, iters, body, (zeros, zeros, zeros))
        pub[pl.ds(0, _LN)] = g
        pub[pl.ds(16, _LN)] = e
        pub[pl.ds(32, _LN)] = s
        pltpu.sync_copy(pub, out_hbm.at[wid])

    return phase1


@functools.lru_cache(maxsize=None)
def _make_fallback(n):
    chunk = n // _NT
    iters = chunk // _LN
    kept = min(_MIN_KEPT, n - 1)
    rank = float(n - 1 - kept)
    mesh = plsc.VectorSubcoreMesh(
        core_axis_name="c", subcore_axis_name="s", num_cores=1)

    @functools.partial(
        pl.kernel,
        out_type=jax.ShapeDtypeStruct((_LN,), jnp.float32),
        mesh=mesh,
        compiler_params=pltpu.CompilerParams(needs_layout_passes=False),
        scratch_types=[
            pltpu.VMEM((chunk,), jnp.float32),
            pltpu.VMEM_SHARED((_NT * 16,), jnp.float32),
            pltpu.VMEM((_NT * 16,), jnp.float32),
            pltpu.VMEM((_LN,), jnp.float32),
            pltpu.VMEM((_LN,), jnp.float32),
        ],
    )
    def fb(l_hbm, out_hbm, buf, stage, stage_l, pub, obuf):
        wid = lax.axis_index("s")
        zeros = jnp.zeros((_LN,), jnp.float32)
        lane = lax.broadcasted_iota(jnp.int32, (_LN,), 0)

        pltpu.sync_copy(l_hbm.at[pl.ds(wid * chunk, chunk)], buf)

        def vchunk(j):
            return buf[pl.ds(pl.multiple_of(j * _LN, _LN), _LN)]

        def vec_to_scalar(v):
            acc = v[0]
            for i in range(1, _LN):
                acc = acc + v[i]
            return acc

        def combine(a):
            pub[pl.ds(0, _LN)] = a
            pltpu.sync_copy(pub, stage.at[pl.ds(wid * 16, _LN)])
            plsc.subcore_barrier()
            pltpu.sync_copy(stage, stage_l)
            ta = zeros
            for t in range(_NT):
                ta = ta + stage_l[pl.ds(t * 16, _LN)]
            plsc.subcore_barrier()
            return vec_to_scalar(ta)

        def bit_round(i, p):
            t_pat = p | lax.shift_left(jnp.int32(1), jnp.int32(30) - i)
            t_val = lax.bitcast_convert_type(t_pat, jnp.float32)

            def cbody(j, acc):
                return acc + jnp.where(vchunk(j) < t_val, 1.0, 0.0)

            cl = lax.fori_loop(0, iters, cbody, zeros)
            total = combine(cl)
            return jnp.where(total <= rank, t_pat, p)

        p = lax.fori_loop(0, 31, bit_round, jnp.int32(0))
        cutoff = lax.bitcast_convert_type(p, jnp.float32)

        def fbody(j, carry):
            g2, s2 = carry
            v = vchunk(j)
            keep = v > cutoff
            return (g2 + jnp.where(keep, 1.0, 0.0),
                    s2 + jnp.where(keep, v, 0.0))

        g2, s2 = lax.fori_loop(0, iters, fbody, (zeros, zeros))
        c_d = combine(g2)
        s_d = combine(s2)

        @pl.when(wid == 0)
        def _():
            obuf[...] = jnp.where(
                lane == 0, s_d, jnp.where(lane == 1, c_d, 0.0))
            pltpu.sync_copy(obuf, out_hbm)

    return fb


def kernel(predict, target):
    ce = _ce_losses(predict, target.astype(jnp.int32))
    flat = ce.reshape(-1)
    n = flat.shape[0]
    parts = _make_phase1(n)(flat)
    c_gt = jnp.sum(parts[:, 0:16])
    c_ge = jnp.sum(parts[:, 16:32])
    s_gt = jnp.sum(parts[:, 32:48])
    kept_f = jnp.float32(min(_MIN_KEPT, n - 1))

    def rare(_):
        out = _make_fallback(n)(flat)
        return out[0], out[1]

    s_sel, c_sel = lax.cond(
        c_ge <= kept_f, rare, lambda _: (s_gt, c_gt), None)
    return jnp.where(c_sel > 0.0, s_sel / jnp.maximum(c_sel, 1.0), 0.0)


# speedup vs baseline: 26.7286x; 1.1152x over previous
import functools
import math

import jax
import jax.numpy as jnp
from jax import lax
from jax.experimental import pallas as pl
from jax.experimental.pallas import tpu as pltpu
from jax.experimental.pallas import tpu_sc as plsc

_MIN_KEPT = 100000
_THRESH = 0.7
_TAU0 = float(-math.log(_THRESH))

_BH = 128
_NC = 2
_NT = 16
_LN = 16


def _ce_body(pred_ref, tgt_ref, out_ref):
    x = pred_ref[0]
    t = tgt_ref[0]
    m = jnp.max(x, axis=0)
    s = jnp.sum(jnp.exp(x - m[None]), axis=0)
    cls = lax.broadcasted_iota(jnp.int32, x.shape, 0)
    xt = jnp.sum(jnp.where(cls == t[None], x, 0.0), axis=0)
    out_ref[0] = (m - xt) + jnp.log(s)


def _ce_losses(predict, target):
    B, C, H, W = predict.shape
    return pl.pallas_call(
        _ce_body,
        grid=(B, H // _BH),
        in_specs=[
            pl.BlockSpec((1, C, _BH, W), lambda b, h: (b, 0, h, 0)),
            pl.BlockSpec((1, _BH, W), lambda b, h: (b, h, 0)),
        ],
        out_specs=pl.BlockSpec((1, _BH, W), lambda b, h: (b, h, 0)),
        out_shape=jax.ShapeDtypeStruct((B, H, W), jnp.float32),
    )(predict, target)


@functools.lru_cache(maxsize=None)
def _make_phase1(n):
    nw = _NC * _NT
    chunk = n // nw
    iters = chunk // _LN
    mesh = plsc.VectorSubcoreMesh(
        core_axis_name="c", subcore_axis_name="s", num_cores=_NC)

    @functools.partial(
        pl.kernel,
        out_type=jax.ShapeDtypeStruct((nw, 48), jnp.float32),
        mesh=mesh,
        compiler_params=pltpu.CompilerParams(needs_layout_passes=False),
        scratch_types=[
            pltpu.VMEM((chunk,), jnp.float32),
            pltpu.VMEM((48,), jnp.float32),
        ],
    )
    def phase1(l_hbm, out_hbm, buf, pub):
        wid = lax.axis_index("s") * _NC + lax.axis_index("c")
        zeros = jnp.zeros((_LN,), jnp.float32)

        pltpu.sync_copy(l_hbm.at[pl.ds(wid * chunk, chunk)], buf)

        def body(j, carry):
            g, e, s = carry
            v = buf[pl.ds(pl.multiple_of(j * _LN, _LN), _LN)]
            g = g + jnp.where(v > _TAU0, 1.0, 0.0)
            e = e + jnp.where(v >= _TAU0, 1.0, 0.0)
            s = s + jnp.where(v > _TAU0, v, 0.0)
            return g, e, s

        g, e, s = lax.fori_loop(0, iters, body, (zeros, zeros, zeros))
        pub[pl.ds(0, _LN)] = g
        pub[pl.ds(16, _LN)] = e
        pub[pl.ds(32, _LN)] = s
        pltpu.sync_copy(pub, out_hbm.at[wid])

    return phase1


@functools.lru_cache(maxsize=None)
def _make_fallback(n):
    chunk = n // _NT
    iters = chunk // _LN
    kept = min(_MIN_KEPT, n - 1)
    rank = float(n - 1 - kept)
    mesh = plsc.VectorSubcoreMesh(
        core_axis_name="c", subcore_axis_name="s", num_cores=1)

    @functools.partial(
        pl.kernel,
        out_type=jax.ShapeDtypeStruct((_LN,), jnp.float32),
        mesh=mesh,
        compiler_params=pltpu.CompilerParams(needs_layout_passes=False),
        scratch_types=[
            pltpu.VMEM((chunk,), jnp.float32),
            pltpu.VMEM_SHARED((_NT * 16,), jnp.float32),
            pltpu.VMEM((_NT * 16,), jnp.float32),
            pltpu.VMEM((_LN,), jnp.float32),
            pltpu.VMEM((_LN,), jnp.float32),
        ],
    )
    def fb(l_hbm, out_hbm, buf, stage, stage_l, pub, obuf):
        wid = lax.axis_index("s")
        zeros = jnp.zeros((_LN,), jnp.float32)
        lane = lax.broadcasted_iota(jnp.int32, (_LN,), 0)

        pltpu.sync_copy(l_hbm.at[pl.ds(wid * chunk, chunk)], buf)

        def vchunk(j):
            return buf[pl.ds(pl.multiple_of(j * _LN, _LN), _LN)]

        def vec_to_scalar(v):
            acc = v[0]
            for i in range(1, _LN):
                acc = acc + v[i]
            return acc

        def combine(a):
            pub[pl.ds(0, _LN)] = a
            pltpu.sync_copy(pub, stage.at[pl.ds(wid * 16, _LN)])
            plsc.subcore_barrier()
            pltpu.sync_copy(stage, stage_l)
            ta = zeros
            for t in range(_NT):
                ta = ta + stage_l[pl.ds(t * 16, _LN)]
            plsc.subcore_barrier()
            return vec_to_scalar(ta)

        def bit_round(i, p):
            t_pat = p | lax.shift_left(jnp.int32(1), jnp.int32(30) - i)
            t_val = lax.bitcast_convert_type(t_pat, jnp.float32)

            def cbody(j, acc):
                return acc + jnp.where(vchunk(j) < t_val, 1.0, 0.0)

            cl = lax.fori_loop(0, iters, cbody, zeros)
            total = combine(cl)
            return jnp.where(total <= rank, t_pat, p)

        p = lax.fori_loop(0, 31, bit_round, jnp.int32(0))
        cutoff = lax.bitcast_convert_type(p, jnp.float32)

        def fbody(j, carry):
            g2, s2 = carry
            v = vchunk(j)
            keep = v > cutoff
            return (g2 + jnp.where(keep, 1.0, 0.0),
                    s2 + jnp.where(keep, v, 0.0))

        g2, s2 = lax.fori_loop(0, iters, fbody, (zeros, zeros))
        c_d = combine(g2)
        s_d = combine(s2)

        @pl.when(wid == 0)
        def _():
            obuf[...] = jnp.where(
                lane == 0, s_d, jnp.where(lane == 1, c_d, 0.0))
            pltpu.sync_copy(obuf, out_hbm)

    return fb


def kernel(predict, target):
    ce = _ce_losses(predict, target.astype(jnp.int32))
    flat = ce.reshape(-1)
    n = flat.shape[0]
    parts = _make_phase1(n)(flat)
    c_gt = jnp.sum(parts[:, 0:16])
    c_ge = jnp.sum(parts[:, 16:32])
    s_gt = jnp.sum(parts[:, 32:48])
    kept_f = jnp.float32(min(_MIN_KEPT, n - 1))

    def rare(_):
        out = _make_fallback(n)(flat)
        return out[0], out[1]

    s_sel, c_sel = lax.cond(
        c_ge <= kept_f, rare, lambda _: (s_gt, c_gt), None)
    return jnp.where(c_sel > 0.0, s_sel / jnp.maximum(c_sel, 1.0), 0.0)


# speedup vs baseline: 27.8174x; 1.0407x over previous
import functools
import math

import jax
import jax.numpy as jnp
from jax import lax
from jax.experimental import pallas as pl
from jax.experimental.pallas import tpu as pltpu
from jax.experimental.pallas import tpu_sc as plsc

_MIN_KEPT = 100000
_THRESH = 0.7
_TAU0 = float(-math.log(_THRESH))

_BH = 256
_NC = 2
_NT = 16
_LN = 16


def _ce_body(pred_ref, tgt_ref, out_ref):
    x = pred_ref[0]
    t = tgt_ref[0]
    m = jnp.max(x, axis=0)
    s = jnp.sum(jnp.exp(x - m[None]), axis=0)
    cls = lax.broadcasted_iota(jnp.int32, x.shape, 0)
    xt = jnp.sum(jnp.where(cls == t[None], x, 0.0), axis=0)
    out_ref[0] = (m - xt) + jnp.log(s)


def _ce_losses(predict, target):
    B, C, H, W = predict.shape
    return pl.pallas_call(
        _ce_body,
        grid=(B, H // _BH),
        in_specs=[
            pl.BlockSpec((1, C, _BH, W), lambda b, h: (b, 0, h, 0)),
            pl.BlockSpec((1, _BH, W), lambda b, h: (b, h, 0)),
        ],
        out_specs=pl.BlockSpec((1, _BH, W), lambda b, h: (b, h, 0)),
        out_shape=jax.ShapeDtypeStruct((B, H, W), jnp.float32),
    )(predict, target)


@functools.lru_cache(maxsize=None)
def _make_phase1(n):
    nw = _NC * _NT
    chunk = n // nw
    iters = chunk // _LN
    mesh = plsc.VectorSubcoreMesh(
        core_axis_name="c", subcore_axis_name="s", num_cores=_NC)

    @functools.partial(
        pl.kernel,
        out_type=jax.ShapeDtypeStruct((nw, 48), jnp.float32),
        mesh=mesh,
        compiler_params=pltpu.CompilerParams(needs_layout_passes=False),
        scratch_types=[
            pltpu.VMEM((chunk,), jnp.float32),
            pltpu.VMEM((48,), jnp.float32),
        ],
    )
    def phase1(l_hbm, out_hbm, buf, pub):
        wid = lax.axis_index("s") * _NC + lax.axis_index("c")
        zeros = jnp.zeros((_LN,), jnp.float32)

        pltpu.sync_copy(l_hbm.at[pl.ds(wid * chunk, chunk)], buf)

        def body(j, carry):
            g, e, s = carry
            v = buf[pl.ds(pl.multiple_of(j * _LN, _LN), _LN)]
            g = g + jnp.where(v > _TAU0, 1.0, 0.0)
            e = e + jnp.where(v >= _TAU0, 1.0, 0.0)
            s = s + jnp.where(v > _TAU0, v, 0.0)
            return g, e, s

        g, e, s = lax.fori_loop(0, iters, body, (zeros, zeros, zeros))
        pub[pl.ds(0, _LN)] = g
        pub[pl.ds(16, _LN)] = e
        pub[pl.ds(32, _LN)] = s
        pltpu.sync_copy(pub, out_hbm.at[wid])

    return phase1


@functools.lru_cache(maxsize=None)
def _make_fallback(n):
    chunk = n // _NT
    iters = chunk // _LN
    kept = min(_MIN_KEPT, n - 1)
    rank = float(n - 1 - kept)
    mesh = plsc.VectorSubcoreMesh(
        core_axis_name="c", subcore_axis_name="s", num_cores=1)

    @functools.partial(
        pl.kernel,
        out_type=jax.ShapeDtypeStruct((_LN,), jnp.float32),
        mesh=mesh,
        compiler_params=pltpu.CompilerParams(needs_layout_passes=False),
        scratch_types=[
            pltpu.VMEM((chunk,), jnp.float32),
            pltpu.VMEM_SHARED((_NT * 16,), jnp.float32),
            pltpu.VMEM((_NT * 16,), jnp.float32),
            pltpu.VMEM((_LN,), jnp.float32),
            pltpu.VMEM((_LN,), jnp.float32),
        ],
    )
    def fb(l_hbm, out_hbm, buf, stage, stage_l, pub, obuf):
        wid = lax.axis_index("s")
        zeros = jnp.zeros((_LN,), jnp.float32)
        lane = lax.broadcasted_iota(jnp.int32, (_LN,), 0)

        pltpu.sync_copy(l_hbm.at[pl.ds(wid * chunk, chunk)], buf)

        def vchunk(j):
            return buf[pl.ds(pl.multiple_of(j * _LN, _LN), _LN)]

        def vec_to_scalar(v):
            acc = v[0]
            for i in range(1, _LN):
                acc = acc + v[i]
            return acc

        def combine(a):
            pub[pl.ds(0, _LN)] = a
            pltpu.sync_copy(pub, stage.at[pl.ds(wid * 16, _LN)])
            plsc.subcore_barrier()
            pltpu.sync_copy(stage, stage_l)
            ta = zeros
            for t in range(_NT):
                ta = ta + stage_l[pl.ds(t * 16, _LN)]
            plsc.subcore_barrier()
            return vec_to_scalar(ta)

        def bit_round(i, p):
            t_pat = p | lax.shift_left(jnp.int32(1), jnp.int32(30) - i)
            t_val = lax.bitcast_convert_type(t_pat, jnp.float32)

            def cbody(j, acc):
                return acc + jnp.where(vchunk(j) < t_val, 1.0, 0.0)

            cl = lax.fori_loop(0, iters, cbody, zeros)
            total = combine(cl)
            return jnp.where(total <= rank, t_pat, p)

        p = lax.fori_loop(0, 31, bit_round, jnp.int32(0))
        cutoff = lax.bitcast_convert_type(p, jnp.float32)

        def fbody(j, carry):
            g2, s2 = carry
            v = vchunk(j)
            keep = v > cutoff
            return (g2 + jnp.where(keep, 1.0, 0.0),
                    s2 + jnp.where(keep, v, 0.0))

        g2, s2 = lax.fori_loop(0, iters, fbody, (zeros, zeros))
        c_d = combine(g2)
        s_d = combine(s2)

        @pl.when(wid == 0)
        def _():
            obuf[...] = jnp.where(
                lane == 0, s_d, jnp.where(lane == 1, c_d, 0.0))
            pltpu.sync_copy(obuf, out_hbm)

    return fb


def kernel(predict, target):
    ce = _ce_losses(predict, target.astype(jnp.int32))
    flat = ce.reshape(-1)
    n = flat.shape[0]
    parts = _make_phase1(n)(flat)
    c_gt = jnp.sum(parts[:, 0:16])
    c_ge = jnp.sum(parts[:, 16:32])
    s_gt = jnp.sum(parts[:, 32:48])
    kept_f = jnp.float32(min(_MIN_KEPT, n - 1))

    def rare(_):
        out = _make_fallback(n)(flat)
        return out[0], out[1]

    s_sel, c_sel = lax.cond(
        c_ge <= kept_f, rare, lambda _: (s_gt, c_gt), None)
    return jnp.where(c_sel > 0.0, s_sel / jnp.maximum(c_sel, 1.0), 0.0)
